# baseline (speedup 1.0000x reference)
import jax
import jax.numpy as jnp
from jax import lax
from jax.experimental import pallas as pl
from jax.experimental.pallas import tpu as pltpu
from jax.experimental.pallas import tpu_sc as plsc

QSIZE = 65536
DIM = 128
B = 1024
NW = 32
STRIPE = QSIZE // NW
C = 512
NCHUNK = STRIPE // C


def _sc_body(queue_hbm, key_hbm, ptr_hbm, out_hbm, buf_v, ptr_v):
    wid = lax.axis_index("s") * 2 + lax.axis_index("c")
    base = wid * STRIPE
    pltpu.sync_copy(ptr_hbm, ptr_v)
    p = ptr_v[...][0]
    for k in range(NCHUNK):
        s = pl.multiple_of(base + k * C, C)
        in_win = jnp.logical_and(s >= p, s < p + B)

        @pl.when(in_win)
        def _():
            pltpu.sync_copy(key_hbm.at[pl.ds(pl.multiple_of(s - p, C), C)], buf_v)

        @pl.when(jnp.logical_not(in_win))
        def _():
            pltpu.sync_copy(queue_hbm.at[pl.ds(s, C)], buf_v)

        pltpu.sync_copy(buf_v, out_hbm.at[pl.ds(s, C)])


def kernel(queue, key_batch, queue_ptr):
    ptr = jnp.full((16,), queue_ptr, jnp.int32)
    mesh = plsc.VectorSubcoreMesh(core_axis_name="c", subcore_axis_name="s")
    f = pl.kernel(
        _sc_body,
        out_type=jax.ShapeDtypeStruct((QSIZE, DIM), jnp.float32),
        mesh=mesh,
        scratch_types=[
            pltpu.VMEM((C, DIM), jnp.float32),
            pltpu.VMEM((16,), jnp.int32),
        ],
    )
    return f(queue, key_batch, ptr)


# speedup vs baseline: 1.0390x; 1.0390x over previous
import jax
import jax.numpy as jnp
from jax import lax
from jax.experimental import pallas as pl
from jax.experimental.pallas import tpu as pltpu
from jax.experimental.pallas import tpu_sc as plsc

QSIZE = 65536
DIM = 128
B = 1024
NW = 32
STRIPE = QSIZE // NW
C = 256
NCHUNK = STRIPE // C


def _sc_body(queue_hbm, key_hbm, ptr_hbm, out_hbm, buf0, buf1, ptr_v, sems):
    wid = lax.axis_index("s") * 2 + lax.axis_index("c")
    base = wid * STRIPE
    pltpu.sync_copy(ptr_hbm, ptr_v)
    p = ptr_v[...][0]
    bufs = (buf0, buf1)

    def chunk_start(s):
        return pl.multiple_of(base + s * C, C)

    def start_in(k):
        s = chunk_start(k)
        buf = bufs[k % 2]
        in_sem = sems.at[k % 2]
        in_win = jnp.logical_and(s >= p, s < p + B)

        @pl.when(in_win)
        def _():
            pltpu.make_async_copy(
                key_hbm.at[pl.ds(pl.multiple_of(s - p, C), C)], buf, in_sem
            ).start()

        @pl.when(jnp.logical_not(in_win))
        def _():
            pltpu.make_async_copy(queue_hbm.at[pl.ds(s, C)], buf, in_sem).start()

    def wait_in(k):
        s = chunk_start(k)
        pltpu.make_async_copy(
            queue_hbm.at[pl.ds(s, C)], bufs[k % 2], sems.at[k % 2]
        ).wait()

    def out_copy(k):
        s = chunk_start(k)
        return pltpu.make_async_copy(
            bufs[k % 2], out_hbm.at[pl.ds(s, C)], sems.at[2 + k % 2]
        )

    start_in(0)
    for k in range(NCHUNK):
        if k + 1 < NCHUNK:
            if k - 1 >= 0:
                out_copy(k - 1).wait()
            start_in(k + 1)
        wait_in(k)
        out_copy(k).start()
    if NCHUNK >= 2:
        out_copy(NCHUNK - 2).wait()
    out_copy(NCHUNK - 1).wait()


def kernel(queue, key_batch, queue_ptr):
    ptr = jnp.full((16,), queue_ptr, jnp.int32)
    mesh = plsc.VectorSubcoreMesh(core_axis_name="c", subcore_axis_name="s")
    f = pl.kernel(
        _sc_body,
        out_type=jax.ShapeDtypeStruct((QSIZE, DIM), jnp.float32),
        mesh=mesh,
        scratch_types=[
            pltpu.VMEM((C, DIM), jnp.float32),
            pltpu.VMEM((C, DIM), jnp.float32),
            pltpu.VMEM((16,), jnp.int32),
            pltpu.SemaphoreType.DMA((4,)),
        ],
    )
    return f(queue, key_batch, ptr)
